# baseline (speedup 1.0000x reference)
import functools

import jax
import jax.numpy as jnp
from jax import lax
from jax.experimental import pallas as pl
from jax.experimental.pallas import tpu as pltpu
from jax.experimental.pallas import tpu_sc as plsc

N = 10000
E = 320000
D = 128
H = 64
G = 8

NC = 2
NS = 16
NW = NC * NS
CH = 128
NCHUNK = 80
E_PAD = NW * NCHUNK * CH
N_ACC = 10240
STRIPE = N_ACC // NS
PAD_ROWS = N_ACC - N
DEGW = 16
NBUF = 8

BLK = 1024
NBLK = N_ACC // BLK

_mesh = plsc.VectorSubcoreMesh(core_axis_name="c", subcore_axis_name="s")

_DOT = dict(precision=lax.Precision.HIGHEST, preferred_element_type=jnp.float32)

_SC_PARAMS = pltpu.CompilerParams(use_tc_tiling_on_sc=False)


def _sc_degree(dst_idx):
    ones = jnp.ones((CH, DEGW), jnp.float32)
    zeros = jnp.zeros((STRIPE, DEGW), jnp.float32)

    @functools.partial(
        pl.kernel,
        mesh=_mesh,
        out_type=jax.ShapeDtypeStruct((NC * N_ACC, DEGW), jnp.float32),
        compiler_params=_SC_PARAMS,
        scratch_types=[
            pltpu.VMEM((NCHUNK, CH), jnp.int32),
            pltpu.VMEM((CH, DEGW), jnp.float32),
            pltpu.VMEM_SHARED((N_ACC, DEGW), jnp.float32),
        ],
    )
    def deg_kernel(dst_hbm, ones_hbm, zeros_hbm, out_hbm, idx_v, ones_v, acc):
        cid = lax.axis_index("c")
        sid = lax.axis_index("s")
        wid = sid * NC + cid
        pltpu.sync_copy(dst_hbm.at[wid], idx_v)
        pltpu.sync_copy(ones_hbm, ones_v)
        pltpu.sync_copy(zeros_hbm, acc.at[pl.ds(sid * STRIPE, STRIPE)])
        plsc.subcore_barrier()

        @pl.loop(0, NCHUNK)
        def _(j):
            pltpu.sync_copy(ones_v, acc.at[idx_v.at[j]], add=True)

        plsc.subcore_barrier()
        base = cid * N_ACC + sid * STRIPE
        pltpu.sync_copy(acc.at[pl.ds(sid * STRIPE, STRIPE)],
                        out_hbm.at[pl.ds(base, STRIPE)])

    return deg_kernel(dst_idx, ones, zeros)


def _sc_mp(hs, src_idx, dst_idx):
    zeros = jnp.zeros((STRIPE, H), jnp.float32)

    @functools.partial(
        pl.kernel,
        mesh=_mesh,
        out_type=jax.ShapeDtypeStruct((NC * N_ACC, H), jnp.float32),
        compiler_params=_SC_PARAMS,
        scratch_types=[
            pltpu.VMEM((NCHUNK, CH), jnp.int32),
            pltpu.VMEM((NCHUNK, CH), jnp.int32),
            pltpu.VMEM((NBUF, CH, H), jnp.float32),
            pltpu.VMEM_SHARED((N_ACC, H), jnp.float32),
        ] + [pltpu.SemaphoreType.DMA] * (2 * NBUF),
    )
    def mp_kernel(hs_hbm, src_hbm, dst_hbm, zeros_hbm, out_hbm,
                  idxs_v, idxd_v, bufs, acc, *sems):
        gsem, ssem = sems[:NBUF], sems[NBUF:]
        cid = lax.axis_index("c")
        sid = lax.axis_index("s")
        wid = sid * NC + cid
        pltpu.sync_copy(src_hbm.at[wid], idxs_v)
        pltpu.sync_copy(dst_hbm.at[wid], idxd_v)
        pltpu.sync_copy(zeros_hbm, acc.at[pl.ds(sid * STRIPE, STRIPE)])
        plsc.subcore_barrier()

        for b in range(NBUF):
            pltpu.async_copy(hs_hbm.at[idxs_v.at[b]], bufs.at[b], gsem[b])

        @pl.loop(0, NCHUNK, step=NBUF)
        def _(j):
            for b in range(NBUF):
                pltpu.make_async_copy(
                    hs_hbm.at[pl.ds(0, CH)], bufs.at[b], gsem[b]).wait()
                pltpu.async_copy(
                    bufs.at[b], acc.at[idxd_v.at[j + b]], ssem[b], add=True)
            for b in range(NBUF):
                pltpu.make_async_copy(
                    hs_hbm.at[pl.ds(0, CH)], bufs.at[b], ssem[b]).wait()

                @pl.when(j + NBUF + b < NCHUNK)
                def _():
                    pltpu.async_copy(
                        hs_hbm.at[idxs_v.at[j + NBUF + b]], bufs.at[b], gsem[b])

        plsc.subcore_barrier()
        base = cid * N_ACC + sid * STRIPE
        pltpu.sync_copy(acc.at[pl.ds(sid * STRIPE, STRIPE)],
                        out_hbm.at[pl.ds(base, STRIPE)])

    return mp_kernel(hs, src_idx, dst_idx, zeros)


def _tc_matmul(x, w):
    def body(x_ref, w_ref, o_ref):
        o_ref[...] = lax.dot_general(x_ref[...], w_ref[...],
                                     (((1,), (0,)), ((), ())), **_DOT)

    return pl.pallas_call(
        body,
        grid=(NBLK,),
        in_specs=[pl.BlockSpec((BLK, x.shape[1]), lambda i: (i, 0)),
                  pl.BlockSpec(w.shape, lambda i: (0, 0))],
        out_specs=pl.BlockSpec((BLK, w.shape[1]), lambda i: (i, 0)),
        out_shape=jax.ShapeDtypeStruct((N_ACC, w.shape[1]), jnp.float32),
    )(x, w)


def _part_specs():
    return [pl.BlockSpec((BLK, H), lambda i: (i, 0)),
            pl.BlockSpec((BLK, H), lambda i: (i + NBLK, 0))]


def _tc_finish1(xw, degs):
    def body(xw_ref, d0_ref, d1_ref, hs_ref, dinv_ref):
        deg = d0_ref[:, 0:1] + d1_ref[:, 0:1] + 1.0
        dinv = lax.rsqrt(jnp.maximum(deg, 1.0))
        dinv_ref[...] = dinv
        hs_ref[...] = xw_ref[...] * dinv

    return pl.pallas_call(
        body,
        grid=(NBLK,),
        in_specs=[pl.BlockSpec((BLK, H), lambda i: (i, 0)),
                  pl.BlockSpec((BLK, DEGW), lambda i: (i, 0)),
                  pl.BlockSpec((BLK, DEGW), lambda i: (i + NBLK, 0))],
        out_specs=[pl.BlockSpec((BLK, H), lambda i: (i, 0)),
                   pl.BlockSpec((BLK, 1), lambda i: (i, 0))],
        out_shape=[jax.ShapeDtypeStruct((N_ACC, H), jnp.float32),
                   jax.ShapeDtypeStruct((N_ACC, 1), jnp.float32)],
    )(xw, degs, degs)


def _tc_combine(p, hs, dinv, b, w_next):
    def body(p0_ref, p1_ref, hs_ref, dinv_ref, b_ref, w_ref, o_ref):
        s = p0_ref[...] + p1_ref[...] + hs_ref[...]
        t = jnp.maximum(dinv_ref[...] * s + b_ref[...], 0.0)
        o_ref[...] = dinv_ref[...] * lax.dot_general(
            t, w_ref[...], (((1,), (0,)), ((), ())), **_DOT)

    return pl.pallas_call(
        body,
        grid=(NBLK,),
        in_specs=_part_specs() + [
            pl.BlockSpec((BLK, H), lambda i: (i, 0)),
            pl.BlockSpec((BLK, 1), lambda i: (i, 0)),
            pl.BlockSpec((1, H), lambda i: (0, 0)),
            pl.BlockSpec((H, H), lambda i: (0, 0))],
        out_specs=pl.BlockSpec((BLK, H), lambda i: (i, 0)),
        out_shape=jax.ShapeDtypeStruct((N_ACC, H), jnp.float32),
    )(p, p, hs, dinv, b, w_next)


def _tc_final(p, hs, dinv, b, batch2d, wl, bl):
    def body(p0_ref, p1_ref, hs_ref, dinv_ref, b_ref, bat_ref,
             wl_ref, bl_ref, o_ref, sums, cnt):
        i = pl.program_id(0)

        @pl.when(i == 0)
        def _():
            sums[...] = jnp.zeros_like(sums)
            cnt[...] = jnp.zeros_like(cnt)

        s = p0_ref[...] + p1_ref[...] + hs_ref[...]
        h3 = dinv_ref[...] * s + b_ref[...]
        gids = lax.broadcasted_iota(jnp.int32, (BLK, G), 1)
        mask = (bat_ref[...] == gids).astype(jnp.float32)
        sums[...] += lax.dot_general(mask, h3, (((0,), (0,)), ((), ())), **_DOT)
        cnt[...] += jnp.sum(mask, axis=0)[:, None]

        @pl.when(i == NBLK - 1)
        def _():
            pooled = sums[...] / jnp.maximum(cnt[...], 1.0)
            o_ref[...] = lax.dot_general(
                pooled, wl_ref[...], (((1,), (0,)), ((), ())), **_DOT) + bl_ref[...]

    return pl.pallas_call(
        body,
        grid=(NBLK,),
        in_specs=_part_specs() + [
            pl.BlockSpec((BLK, H), lambda i: (i, 0)),
            pl.BlockSpec((BLK, 1), lambda i: (i, 0)),
            pl.BlockSpec((1, H), lambda i: (0, 0)),
            pl.BlockSpec((BLK, 1), lambda i: (i, 0)),
            pl.BlockSpec((H, 1), lambda i: (0, 0)),
            pl.BlockSpec((1, 1), lambda i: (0, 0))],
        out_specs=pl.BlockSpec((G, 1), lambda i: (0, 0)),
        out_shape=jax.ShapeDtypeStruct((G, 1), jnp.float32),
        scratch_shapes=[pltpu.VMEM((G, H), jnp.float32),
                        pltpu.VMEM((G, 1), jnp.float32)],
    )(p, p, hs, dinv, b, batch2d, wl, bl)


def kernel(x, edge_index, batch, W1, b1, W2, b2, W3, b3, Wl, bl):
    src = edge_index[0]
    dst = edge_index[1]
    pad = E_PAD - E
    pad_src = jnp.zeros((pad,), jnp.int32)
    pad_dst = N + (jnp.arange(pad, dtype=jnp.int32) % PAD_ROWS)
    src_p = jnp.concatenate([src, pad_src]).reshape(NW, NCHUNK, CH)
    dst_p = jnp.concatenate([dst, pad_dst]).reshape(NW, NCHUNK, CH)
    x_p = jnp.concatenate([x, jnp.zeros((PAD_ROWS, D), jnp.float32)])
    batch_p = jnp.concatenate(
        [batch, jnp.full((PAD_ROWS,), G, jnp.int32)]).reshape(N_ACC, 1)

    degs = _sc_degree(dst_p)
    xw = _tc_matmul(x_p, W1)
    hs, dinv = _tc_finish1(xw, degs)

    for b, w_next in ((b1, W2), (b2, W3)):
        p = _sc_mp(hs, src_p, dst_p)
        hs = _tc_combine(p, hs, dinv, b.reshape(1, H), w_next)

    p = _sc_mp(hs, src_p, dst_p)
    return _tc_final(p, hs, dinv, b3.reshape(1, H),
                     batch_p, Wl, bl.reshape(1, 1))


# speedup vs baseline: 1.7027x; 1.7027x over previous
import functools

import jax
import jax.numpy as jnp
from jax import lax
from jax.experimental import pallas as pl
from jax.experimental.pallas import tpu as pltpu
from jax.experimental.pallas import tpu_sc as plsc

N = 10000
E = 320000
D = 128
H = 64
H2 = H // 2
G = 8

NC = 2
NS = 16
NW = NC * NS
CH = 128
NCHUNK = 80
E_PAD = NW * NCHUNK * CH
N_ACC = 10240
STRIPE = N_ACC // NS
PAD_ROWS = N_ACC - N
DEGW = 16
NBUF = 8

BLK = 1024
NBLK = N_ACC // BLK

_mesh = plsc.VectorSubcoreMesh(core_axis_name="c", subcore_axis_name="s")

_DOT = dict(precision=lax.Precision.DEFAULT, preferred_element_type=jnp.float32)
_DOT_HI = dict(precision=lax.Precision.HIGHEST, preferred_element_type=jnp.float32)

_SC_PARAMS = pltpu.CompilerParams(use_tc_tiling_on_sc=False)


def _sc_degree(dst_idx):
    ones = jnp.ones((CH, DEGW), jnp.float32)
    zeros = jnp.zeros((STRIPE, DEGW), jnp.float32)

    @functools.partial(
        pl.kernel,
        mesh=_mesh,
        out_type=jax.ShapeDtypeStruct((NC * N_ACC, DEGW), jnp.float32),
        compiler_params=_SC_PARAMS,
        scratch_types=[
            pltpu.VMEM((NCHUNK, CH), jnp.int32),
            pltpu.VMEM((CH, DEGW), jnp.float32),
            pltpu.VMEM_SHARED((N_ACC, DEGW), jnp.float32),
        ],
    )
    def deg_kernel(dst_hbm, ones_hbm, zeros_hbm, out_hbm, idx_v, ones_v, acc):
        cid = lax.axis_index("c")
        sid = lax.axis_index("s")
        wid = sid * NC + cid
        pltpu.sync_copy(dst_hbm.at[wid], idx_v)
        pltpu.sync_copy(ones_hbm, ones_v)
        pltpu.sync_copy(zeros_hbm, acc.at[pl.ds(sid * STRIPE, STRIPE)])
        plsc.subcore_barrier()

        @pl.loop(0, NCHUNK)
        def _(j):
            pltpu.sync_copy(ones_v, acc.at[idx_v.at[j]], add=True)

        plsc.subcore_barrier()
        base = cid * N_ACC + sid * STRIPE
        pltpu.sync_copy(acc.at[pl.ds(sid * STRIPE, STRIPE)],
                        out_hbm.at[pl.ds(base, STRIPE)])

    return deg_kernel(dst_idx, ones, zeros)


def _sc_mp(hs_half, src_idx, dst_idx):
    zeros = jnp.zeros((STRIPE, H2), jnp.float32)

    @functools.partial(
        pl.kernel,
        mesh=_mesh,
        out_type=jax.ShapeDtypeStruct((NC * N_ACC, H2), jnp.float32),
        compiler_params=_SC_PARAMS,
        scratch_types=[
            pltpu.VMEM((NCHUNK, CH), jnp.int32),
            pltpu.VMEM((NCHUNK, CH), jnp.int32),
            pltpu.VMEM((NBUF, CH, H2), jnp.float32),
            pltpu.VMEM_SHARED((N_ACC, H2), jnp.float32),
            pltpu.VMEM_SHARED((N_ACC, H2), jnp.float32),
        ] + [pltpu.SemaphoreType.DMA] * (2 * NBUF),
    )
    def mp_kernel(hs_hbm, src_hbm, dst_hbm, zeros_hbm, out_hbm,
                  idxs_v, idxd_v, bufs, acc, hs_sh, *sems):
        gsem, ssem = sems[:NBUF], sems[NBUF:]
        cid = lax.axis_index("c")
        sid = lax.axis_index("s")
        wid = sid * NC + cid
        pltpu.sync_copy(src_hbm.at[wid], idxs_v)
        pltpu.sync_copy(dst_hbm.at[wid], idxd_v)
        pltpu.sync_copy(zeros_hbm, acc.at[pl.ds(sid * STRIPE, STRIPE)])
        pltpu.sync_copy(hs_hbm.at[pl.ds(sid * STRIPE, STRIPE)],
                        hs_sh.at[pl.ds(sid * STRIPE, STRIPE)])
        plsc.subcore_barrier()

        for b in range(NBUF):
            pltpu.async_copy(hs_sh.at[idxs_v.at[b]], bufs.at[b], gsem[b])

        @pl.loop(0, NCHUNK, step=NBUF)
        def _(j):
            for b in range(NBUF):
                pltpu.make_async_copy(
                    hs_hbm.at[pl.ds(0, CH)], bufs.at[b], gsem[b]).wait()
                pltpu.async_copy(
                    bufs.at[b], acc.at[idxd_v.at[j + b]], ssem[b], add=True)
            for b in range(NBUF):
                pltpu.make_async_copy(
                    hs_hbm.at[pl.ds(0, CH)], bufs.at[b], ssem[b]).wait()

                @pl.when(j + NBUF + b < NCHUNK)
                def _():
                    pltpu.async_copy(
                        hs_sh.at[idxs_v.at[j + NBUF + b]], bufs.at[b], gsem[b])

        plsc.subcore_barrier()
        base = cid * N_ACC + sid * STRIPE
        pltpu.sync_copy(acc.at[pl.ds(sid * STRIPE, STRIPE)],
                        out_hbm.at[pl.ds(base, STRIPE)])

    return mp_kernel(hs_half, src_idx, dst_idx, zeros)


def _tc_matmul(x, w):
    def body(x_ref, w_ref, o_ref):
        o_ref[...] = lax.dot_general(x_ref[...], w_ref[...],
                                     (((1,), (0,)), ((), ())), **_DOT)

    return pl.pallas_call(
        body,
        grid=(NBLK,),
        in_specs=[pl.BlockSpec((BLK, x.shape[1]), lambda i: (i, 0)),
                  pl.BlockSpec(w.shape, lambda i: (0, 0))],
        out_specs=pl.BlockSpec((BLK, w.shape[1]), lambda i: (i, 0)),
        out_shape=jax.ShapeDtypeStruct((N_ACC, w.shape[1]), jnp.float32),
    )(x, w)


def _half_specs(i_off=0):
    return [pl.BlockSpec((BLK, H2), lambda i: (i, 0)),
            pl.BlockSpec((BLK, H2), lambda i: (i + NBLK, 0))]


def _tc_finish1(xw, degs):
    def body(xw_ref, d0_ref, d1_ref, hsa_ref, hsb_ref, dinv_ref):
        deg = d0_ref[:, 0:1] + d1_ref[:, 0:1] + 1.0
        dinv = lax.rsqrt(jnp.maximum(deg, 1.0))
        dinv_ref[...] = dinv
        hs = xw_ref[...] * dinv
        hsa_ref[...] = hs[:, :H2]
        hsb_ref[...] = hs[:, H2:]

    return pl.pallas_call(
        body,
        grid=(NBLK,),
        in_specs=[pl.BlockSpec((BLK, H), lambda i: (i, 0)),
                  pl.BlockSpec((BLK, DEGW), lambda i: (i, 0)),
                  pl.BlockSpec((BLK, DEGW), lambda i: (i + NBLK, 0))],
        out_specs=[pl.BlockSpec((BLK, H2), lambda i: (i, 0)),
                   pl.BlockSpec((BLK, H2), lambda i: (i, 0)),
                   pl.BlockSpec((BLK, 1), lambda i: (i, 0))],
        out_shape=[jax.ShapeDtypeStruct((N_ACC, H2), jnp.float32),
                   jax.ShapeDtypeStruct((N_ACC, H2), jnp.float32),
                   jax.ShapeDtypeStruct((N_ACC, 1), jnp.float32)],
    )(xw, degs, degs)


def _combine_block(pa0, pa1, pb0, pb1, hsa, hsb, dinv, b):
    sa = pa0[...] + pa1[...] + hsa[...]
    sb = pb0[...] + pb1[...] + hsb[...]
    return dinv[...] * jnp.concatenate([sa, sb], axis=1) + b[...]


def _tc_combine(pa, pb, hsa, hsb, dinv, b, w_next):
    def body(pa0, pa1, pb0, pb1, hsa_ref, hsb_ref, dinv_ref, b_ref, w_ref,
             oa_ref, ob_ref):
        t = jnp.maximum(_combine_block(pa0, pa1, pb0, pb1,
                                       hsa_ref, hsb_ref, dinv_ref, b_ref), 0.0)
        u = dinv_ref[...] * lax.dot_general(
            t, w_ref[...], (((1,), (0,)), ((), ())), **_DOT)
        oa_ref[...] = u[:, :H2]
        ob_ref[...] = u[:, H2:]

    return pl.pallas_call(
        body,
        grid=(NBLK,),
        in_specs=_half_specs() + _half_specs() + [
            pl.BlockSpec((BLK, H2), lambda i: (i, 0)),
            pl.BlockSpec((BLK, H2), lambda i: (i, 0)),
            pl.BlockSpec((BLK, 1), lambda i: (i, 0)),
            pl.BlockSpec((1, H), lambda i: (0, 0)),
            pl.BlockSpec((H, H), lambda i: (0, 0))],
        out_specs=[pl.BlockSpec((BLK, H2), lambda i: (i, 0)),
                   pl.BlockSpec((BLK, H2), lambda i: (i, 0))],
        out_shape=[jax.ShapeDtypeStruct((N_ACC, H2), jnp.float32),
                   jax.ShapeDtypeStruct((N_ACC, H2), jnp.float32)],
    )(pa, pa, pb, pb, hsa, hsb, dinv, b, w_next)


def _tc_final(pa, pb, hsa, hsb, dinv, b, batch2d, wl, bl):
    def body(pa0, pa1, pb0, pb1, hsa_ref, hsb_ref, dinv_ref, b_ref, bat_ref,
             wl_ref, bl_ref, o_ref, sums, cnt):
        i = pl.program_id(0)

        @pl.when(i == 0)
        def _():
            sums[...] = jnp.zeros_like(sums)
            cnt[...] = jnp.zeros_like(cnt)

        h3 = _combine_block(pa0, pa1, pb0, pb1, hsa_ref, hsb_ref,
                            dinv_ref, b_ref)
        gids = lax.broadcasted_iota(jnp.int32, (BLK, G), 1)
        mask = (bat_ref[...] == gids).astype(jnp.float32)
        sums[...] += lax.dot_general(mask, h3, (((0,), (0,)), ((), ())),
                                     **_DOT_HI)
        cnt[...] += jnp.sum(mask, axis=0)[:, None]

        @pl.when(i == NBLK - 1)
        def _():
            pooled = sums[...] / jnp.maximum(cnt[...], 1.0)
            o_ref[...] = lax.dot_general(
                pooled, wl_ref[...], (((1,), (0,)), ((), ())), **_DOT) + bl_ref[...]

    return pl.pallas_call(
        body,
        grid=(NBLK,),
        in_specs=_half_specs() + _half_specs() + [
            pl.BlockSpec((BLK, H2), lambda i: (i, 0)),
            pl.BlockSpec((BLK, H2), lambda i: (i, 0)),
            pl.BlockSpec((BLK, 1), lambda i: (i, 0)),
            pl.BlockSpec((1, H), lambda i: (0, 0)),
            pl.BlockSpec((BLK, 1), lambda i: (i, 0)),
            pl.BlockSpec((H, 1), lambda i: (0, 0)),
            pl.BlockSpec((1, 1), lambda i: (0, 0))],
        out_specs=pl.BlockSpec((G, 1), lambda i: (0, 0)),
        out_shape=jax.ShapeDtypeStruct((G, 1), jnp.float32),
        scratch_shapes=[pltpu.VMEM((G, H), jnp.float32),
                        pltpu.VMEM((G, 1), jnp.float32)],
    )(pa, pa, pb, pb, hsa, hsb, dinv, b, batch2d, wl, bl)


def kernel(x, edge_index, batch, W1, b1, W2, b2, W3, b3, Wl, bl):
    src = edge_index[0]
    dst = edge_index[1]
    pad = E_PAD - E
    pad_src = jnp.zeros((pad,), jnp.int32)
    pad_dst = N + (jnp.arange(pad, dtype=jnp.int32) % PAD_ROWS)
    src_p = jnp.concatenate([src, pad_src]).reshape(NW, NCHUNK, CH)
    dst_p = jnp.concatenate([dst, pad_dst]).reshape(NW, NCHUNK, CH)
    x_p = jnp.concatenate([x, jnp.zeros((PAD_ROWS, D), jnp.float32)])
    batch_p = jnp.concatenate(
        [batch, jnp.full((PAD_ROWS,), G, jnp.int32)]).reshape(N_ACC, 1)

    degs = _sc_degree(dst_p)
    xw = _tc_matmul(x_p, W1)
    hsa, hsb, dinv = _tc_finish1(xw, degs)

    for b, w_next in ((b1, W2), (b2, W3)):
        pa = _sc_mp(hsa, src_p, dst_p)
        pb = _sc_mp(hsb, src_p, dst_p)
        hsa, hsb = _tc_combine(pa, pb, hsa, hsb, dinv,
                               b.reshape(1, H), w_next)

    pa = _sc_mp(hsa, src_p, dst_p)
    pb = _sc_mp(hsb, src_p, dst_p)
    return _tc_final(pa, pb, hsa, hsb, dinv, b3.reshape(1, H),
                     batch_p, Wl, bl.reshape(1, 1))


# speedup vs baseline: 1.7081x; 1.0032x over previous
import functools

import jax
import jax.numpy as jnp
from jax import lax
from jax.experimental import pallas as pl
from jax.experimental.pallas import tpu as pltpu
from jax.experimental.pallas import tpu_sc as plsc

N = 10000
E = 320000
D = 128
H = 64
H2 = H // 2
G = 8

NC = 2
NS = 16
NW = NC * NS
CH = 128
NCHUNK = 80
E_PAD = NW * NCHUNK * CH
N_ACC = 10240
STRIPE = N_ACC // NS
PAD_ROWS = N_ACC - N
DEGW = 16
NBUF = 8

BLK = 1024
NBLK = N_ACC // BLK

_mesh = plsc.VectorSubcoreMesh(core_axis_name="c", subcore_axis_name="s")

_DOT = dict(precision=lax.Precision.DEFAULT, preferred_element_type=jnp.float32)
_DOT_HI = dict(precision=lax.Precision.HIGHEST, preferred_element_type=jnp.float32)

_SC_PARAMS = pltpu.CompilerParams(use_tc_tiling_on_sc=False)


def _sc_degree(dst_idx):
    ones = jnp.ones((CH, DEGW), jnp.float32)
    zeros = jnp.zeros((STRIPE, DEGW), jnp.float32)

    @functools.partial(
        pl.kernel,
        mesh=_mesh,
        out_type=jax.ShapeDtypeStruct((NC * N_ACC, DEGW), jnp.float32),
        compiler_params=_SC_PARAMS,
        scratch_types=[
            pltpu.VMEM((NCHUNK, CH), jnp.int32),
            pltpu.VMEM((CH, DEGW), jnp.float32),
            pltpu.VMEM_SHARED((N_ACC, DEGW), jnp.float32),
        ],
    )
    def deg_kernel(dst_hbm, ones_hbm, zeros_hbm, out_hbm, idx_v, ones_v, acc):
        cid = lax.axis_index("c")
        sid = lax.axis_index("s")
        wid = sid * NC + cid
        pltpu.sync_copy(dst_hbm.at[wid], idx_v)
        pltpu.sync_copy(ones_hbm, ones_v)
        pltpu.sync_copy(zeros_hbm, acc.at[pl.ds(sid * STRIPE, STRIPE)])
        plsc.subcore_barrier()

        @pl.loop(0, NCHUNK)
        def _(j):
            pltpu.sync_copy(ones_v, acc.at[idx_v.at[j]], add=True)

        plsc.subcore_barrier()
        base = cid * N_ACC + sid * STRIPE
        pltpu.sync_copy(acc.at[pl.ds(sid * STRIPE, STRIPE)],
                        out_hbm.at[pl.ds(base, STRIPE)])

    return deg_kernel(dst_idx, ones, zeros)


def _sc_mp(hsa, hsb, src_idx, dst_idx):
    zeros = jnp.zeros((STRIPE, H2), jnp.float32)

    @functools.partial(
        pl.kernel,
        mesh=_mesh,
        out_type=[jax.ShapeDtypeStruct((NC * N_ACC, H2), jnp.float32),
                  jax.ShapeDtypeStruct((NC * N_ACC, H2), jnp.float32)],
        compiler_params=_SC_PARAMS,
        scratch_types=[
            pltpu.VMEM((NCHUNK, CH), jnp.int32),
            pltpu.VMEM((NCHUNK, CH), jnp.int32),
            pltpu.VMEM((NBUF, CH, H2), jnp.float32),
            pltpu.VMEM_SHARED((N_ACC, H2), jnp.float32),
            pltpu.VMEM_SHARED((N_ACC, H2), jnp.float32),
            pltpu.VMEM_SHARED((N_ACC, H2), jnp.float32),
        ] + [pltpu.SemaphoreType.DMA] * (2 * NBUF + 1),
    )
    def mp_kernel(hsa_hbm, hsb_hbm, src_hbm, dst_hbm, zeros_hbm,
                  outa_hbm, outb_hbm,
                  idxs_v, idxd_v, bufs, acc, hsa_sh, hsb_sh, *sems):
        gsem, ssem, bsem = sems[:NBUF], sems[NBUF:2 * NBUF], sems[2 * NBUF]
        cid = lax.axis_index("c")
        sid = lax.axis_index("s")
        wid = sid * NC + cid
        tile_rows = pl.ds(sid * STRIPE, STRIPE)
        pltpu.sync_copy(src_hbm.at[wid], idxs_v)
        pltpu.sync_copy(dst_hbm.at[wid], idxd_v)
        pltpu.sync_copy(zeros_hbm, acc.at[tile_rows])
        pltpu.async_copy(hsb_hbm.at[tile_rows], hsb_sh.at[tile_rows], bsem)
        pltpu.sync_copy(hsa_hbm.at[tile_rows], hsa_sh.at[tile_rows])
        plsc.subcore_barrier()

        def run_half(hs_sh):
            for b in range(NBUF):
                pltpu.async_copy(hs_sh.at[idxs_v.at[b]], bufs.at[b], gsem[b])

            @pl.loop(0, NCHUNK, step=NBUF)
            def _(j):
                for b in range(NBUF):
                    pltpu.make_async_copy(
                        hsa_hbm.at[pl.ds(0, CH)], bufs.at[b], gsem[b]).wait()
                    pltpu.async_copy(
                        bufs.at[b], acc.at[idxd_v.at[j + b]], ssem[b], add=True)
                for b in range(NBUF):
                    pltpu.make_async_copy(
                        hsa_hbm.at[pl.ds(0, CH)], bufs.at[b], ssem[b]).wait()

                    @pl.when(j + NBUF + b < NCHUNK)
                    def _():
                        pltpu.async_copy(
                            hs_sh.at[idxs_v.at[j + NBUF + b]], bufs.at[b],
                            gsem[b])

        run_half(hsa_sh)
        plsc.subcore_barrier()
        base = cid * N_ACC + sid * STRIPE
        out_rows = pl.ds(base, STRIPE)
        pltpu.sync_copy(acc.at[tile_rows], outa_hbm.at[out_rows])
        pltpu.sync_copy(zeros_hbm, acc.at[tile_rows])
        pltpu.make_async_copy(
            hsb_hbm.at[tile_rows], hsb_sh.at[tile_rows], bsem).wait()
        plsc.subcore_barrier()

        run_half(hsb_sh)
        plsc.subcore_barrier()
        pltpu.sync_copy(acc.at[tile_rows], outb_hbm.at[out_rows])

    return mp_kernel(hsa, hsb, src_idx, dst_idx, zeros)


def _tc_matmul(x, w):
    def body(x_ref, w_ref, o_ref):
        o_ref[...] = lax.dot_general(x_ref[...], w_ref[...],
                                     (((1,), (0,)), ((), ())), **_DOT)

    return pl.pallas_call(
        body,
        grid=(NBLK,),
        in_specs=[pl.BlockSpec((BLK, x.shape[1]), lambda i: (i, 0)),
                  pl.BlockSpec(w.shape, lambda i: (0, 0))],
        out_specs=pl.BlockSpec((BLK, w.shape[1]), lambda i: (i, 0)),
        out_shape=jax.ShapeDtypeStruct((N_ACC, w.shape[1]), jnp.float32),
    )(x, w)


def _half_specs():
    return [pl.BlockSpec((BLK, H2), lambda i: (i, 0)),
            pl.BlockSpec((BLK, H2), lambda i: (i + NBLK, 0))]


def _tc_finish1(xw, degs):
    def body(xw_ref, d0_ref, d1_ref, hsa_ref, hsb_ref, dinv_ref):
        deg = d0_ref[:, 0:1] + d1_ref[:, 0:1] + 1.0
        dinv = lax.rsqrt(jnp.maximum(deg, 1.0))
        dinv_ref[...] = dinv
        hs = xw_ref[...] * dinv
        hsa_ref[...] = hs[:, :H2]
        hsb_ref[...] = hs[:, H2:]

    return pl.pallas_call(
        body,
        grid=(NBLK,),
        in_specs=[pl.BlockSpec((BLK, H), lambda i: (i, 0)),
                  pl.BlockSpec((BLK, DEGW), lambda i: (i, 0)),
                  pl.BlockSpec((BLK, DEGW), lambda i: (i + NBLK, 0))],
        out_specs=[pl.BlockSpec((BLK, H2), lambda i: (i, 0)),
                   pl.BlockSpec((BLK, H2), lambda i: (i, 0)),
                   pl.BlockSpec((BLK, 1), lambda i: (i, 0))],
        out_shape=[jax.ShapeDtypeStruct((N_ACC, H2), jnp.float32),
                   jax.ShapeDtypeStruct((N_ACC, H2), jnp.float32),
                   jax.ShapeDtypeStruct((N_ACC, 1), jnp.float32)],
    )(xw, degs, degs)


def _combine_block(pa0, pa1, pb0, pb1, hsa, hsb, dinv, b):
    sa = pa0[...] + pa1[...] + hsa[...]
    sb = pb0[...] + pb1[...] + hsb[...]
    return dinv[...] * jnp.concatenate([sa, sb], axis=1) + b[...]


def _tc_combine(pa, pb, hsa, hsb, dinv, b, w_next):
    def body(pa0, pa1, pb0, pb1, hsa_ref, hsb_ref, dinv_ref, b_ref, w_ref,
             oa_ref, ob_ref):
        t = jnp.maximum(_combine_block(pa0, pa1, pb0, pb1,
                                       hsa_ref, hsb_ref, dinv_ref, b_ref), 0.0)
        u = dinv_ref[...] * lax.dot_general(
            t, w_ref[...], (((1,), (0,)), ((), ())), **_DOT)
        oa_ref[...] = u[:, :H2]
        ob_ref[...] = u[:, H2:]

    return pl.pallas_call(
        body,
        grid=(NBLK,),
        in_specs=_half_specs() + _half_specs() + [
            pl.BlockSpec((BLK, H2), lambda i: (i, 0)),
            pl.BlockSpec((BLK, H2), lambda i: (i, 0)),
            pl.BlockSpec((BLK, 1), lambda i: (i, 0)),
            pl.BlockSpec((1, H), lambda i: (0, 0)),
            pl.BlockSpec((H, H), lambda i: (0, 0))],
        out_specs=[pl.BlockSpec((BLK, H2), lambda i: (i, 0)),
                   pl.BlockSpec((BLK, H2), lambda i: (i, 0))],
        out_shape=[jax.ShapeDtypeStruct((N_ACC, H2), jnp.float32),
                   jax.ShapeDtypeStruct((N_ACC, H2), jnp.float32)],
    )(pa, pa, pb, pb, hsa, hsb, dinv, b, w_next)


def _tc_final(pa, pb, hsa, hsb, dinv, b, batch2d, wl, bl):
    def body(pa0, pa1, pb0, pb1, hsa_ref, hsb_ref, dinv_ref, b_ref, bat_ref,
             wl_ref, bl_ref, o_ref, sums, cnt):
        i = pl.program_id(0)

        @pl.when(i == 0)
        def _():
            sums[...] = jnp.zeros_like(sums)
            cnt[...] = jnp.zeros_like(cnt)

        h3 = _combine_block(pa0, pa1, pb0, pb1, hsa_ref, hsb_ref,
                            dinv_ref, b_ref)
        gids = lax.broadcasted_iota(jnp.int32, (BLK, G), 1)
        mask = (bat_ref[...] == gids).astype(jnp.float32)
        sums[...] += lax.dot_general(mask, h3, (((0,), (0,)), ((), ())),
                                     **_DOT_HI)
        cnt[...] += jnp.sum(mask, axis=0)[:, None]

        @pl.when(i == NBLK - 1)
        def _():
            pooled = sums[...] / jnp.maximum(cnt[...], 1.0)
            o_ref[...] = lax.dot_general(
                pooled, wl_ref[...], (((1,), (0,)), ((), ())), **_DOT) + bl_ref[...]

    return pl.pallas_call(
        body,
        grid=(NBLK,),
        in_specs=_half_specs() + _half_specs() + [
            pl.BlockSpec((BLK, H2), lambda i: (i, 0)),
            pl.BlockSpec((BLK, H2), lambda i: (i, 0)),
            pl.BlockSpec((BLK, 1), lambda i: (i, 0)),
            pl.BlockSpec((1, H), lambda i: (0, 0)),
            pl.BlockSpec((BLK, 1), lambda i: (i, 0)),
            pl.BlockSpec((H, 1), lambda i: (0, 0)),
            pl.BlockSpec((1, 1), lambda i: (0, 0))],
        out_specs=pl.BlockSpec((G, 1), lambda i: (0, 0)),
        out_shape=jax.ShapeDtypeStruct((G, 1), jnp.float32),
        scratch_shapes=[pltpu.VMEM((G, H), jnp.float32),
                        pltpu.VMEM((G, 1), jnp.float32)],
    )(pa, pa, pb, pb, hsa, hsb, dinv, b, batch2d, wl, bl)


def kernel(x, edge_index, batch, W1, b1, W2, b2, W3, b3, Wl, bl):
    src = edge_index[0]
    dst = edge_index[1]
    pad = E_PAD - E
    pad_src = jnp.zeros((pad,), jnp.int32)
    pad_dst = N + (jnp.arange(pad, dtype=jnp.int32) % PAD_ROWS)
    src_p = jnp.concatenate([src, pad_src]).reshape(NW, NCHUNK, CH)
    dst_p = jnp.concatenate([dst, pad_dst]).reshape(NW, NCHUNK, CH)
    x_p = jnp.concatenate([x, jnp.zeros((PAD_ROWS, D), jnp.float32)])
    batch_p = jnp.concatenate(
        [batch, jnp.full((PAD_ROWS,), G, jnp.int32)]).reshape(N_ACC, 1)

    degs = _sc_degree(dst_p)
    xw = _tc_matmul(x_p, W1)
    hsa, hsb, dinv = _tc_finish1(xw, degs)

    for b, w_next in ((b1, W2), (b2, W3)):
        pa, pb = _sc_mp(hsa, hsb, src_p, dst_p)
        hsa, hsb = _tc_combine(pa, pb, hsa, hsb, dinv, b.reshape(1, H), w_next)

    pa, pb = _sc_mp(hsa, hsb, src_p, dst_p)
    return _tc_final(pa, pb, hsa, hsb, dinv, b3.reshape(1, H),
                     batch_p, Wl, bl.reshape(1, 1))
